# baseline (speedup 1.0000x reference)
import functools

import jax
import jax.numpy as jnp
from jax import lax
from jax.experimental import pallas as pl
from jax.experimental.pallas import tpu as pltpu
from jax.experimental.pallas import tpu_sc as plsc

NW = 32
CHUNK = 128
NBUF = 4
VT = 256


def _make_convert(V, D):
    n_full = V // VT
    tail = V % VT
    my_n = -(-n_full // NW)
    n_pairs = my_n // 2
    assert my_n % 2 == 1, "schedule below assumes an odd per-worker count"
    mesh = plsc.VectorSubcoreMesh(core_axis_name="c", subcore_axis_name="s")

    @functools.partial(
        pl.kernel,
        mesh=mesh,
        out_type=jax.ShapeDtypeStruct((V * D,), jnp.float32),
        scratch_types=[
            pltpu.VMEM((2 * D, VT), jnp.float32),
            pltpu.VMEM((2 * VT * D,), jnp.float32),
            pltpu.SemaphoreType.DMA((2,)),
            pltpu.SemaphoreType.DMA((2,)),
        ],
        compiler_params=pltpu.CompilerParams(
            use_tc_tiling_on_sc=True, needs_layout_passes=False),
    )
    def convert_kernel(tt_hbm, tail_hbm, out_hbm, in_v, tr_v, isem, osem):
        wid = lax.axis_index("s") * 2 + lax.axis_index("c")
        base = wid * my_n

        lanes_d = lax.iota(jnp.int32, 16) * D

        def v0(s):
            v = jnp.minimum((base + s) * VT, (n_full - 1) * VT)
            return pl.multiple_of(v, VT)

        def start_in(s, b):
            pltpu.async_copy(
                tt_hbm.at[:, pl.ds(v0(s), VT)],
                in_v.at[pl.ds(b * D, D)], isem.at[b])

        def wait_in(b):
            pltpu.make_async_copy(
                tt_hbm.at[:, pl.ds(0, VT)],
                in_v.at[pl.ds(b * D, D)], isem.at[b]).wait()

        def start_out(s, b):
            pltpu.async_copy(
                tr_v.at[pl.ds(b * VT * D, VT * D)],
                out_hbm.at[pl.ds(v0(s) * D, VT * D)], osem.at[b])

        def wait_out(b):
            pltpu.make_async_copy(
                tr_v.at[pl.ds(b * VT * D, VT * D)],
                out_hbm.at[pl.ds(0, VT * D)], osem.at[b]).wait()

        def transpose(b, kmax=VT // 16):
            @plsc.parallel_loop(0, kmax, unroll=2)
            def _(k):
                koff = k * (16 * D)
                for d in range(D):
                    vals = in_v[b * D + d, pl.ds(k * 16, 16)]
                    plsc.store_scatter(
                        tr_v, [lanes_d + koff + (b * VT * D + d)], vals)
            off = b * VT * D + VT * D - 16
            tr_v[pl.ds(off, 16)] = tr_v[pl.ds(off, 16)] + 0.0

        if tail:
            @pl.when(wid == NW - 1)
            def _():
                pltpu.sync_copy(tail_hbm, in_v.at[pl.ds(0, D)])
                @plsc.parallel_loop(0, VT // 16, unroll=2)
                def _(k):
                    koff = k * (16 * D)
                    for d in range(D):
                        vals = in_v[d, pl.ds(k * 16, 16)]
                        plsc.store_scatter(tr_v, [lanes_d + koff + d], vals)
                off = VT * D - 16
                tr_v[pl.ds(off, 16)] = tr_v[pl.ds(off, 16)] + 0.0
                pltpu.sync_copy(
                    tr_v.at[pl.ds(0, VT * D)],
                    out_hbm.at[pl.ds((V - VT) * D, VT * D)])

        start_in(0, 0)
        start_in(1, 1)

        def pair(s2, _):
            s0 = s2 * 2
            s1 = s0 + 1

            wait_in(0)

            @pl.when(s2 >= 1)
            def _():
                wait_out(0)

            transpose(0)
            start_out(s0, 0)

            @pl.when(s0 + 2 < my_n)
            def _():
                start_in(s0 + 2, 0)

            @pl.when(s1 < my_n)
            def _():
                wait_in(1)

                @pl.when(s2 >= 1)
                def _():
                    wait_out(1)

                transpose(1)
                start_out(s1, 1)

                @pl.when(s1 + 2 < my_n)
                def _():
                    start_in(s1 + 2, 1)

            return 0

        lax.fori_loop(0, (my_n + 1) // 2, pair, 0)

        wait_out(0)
        wait_out(1)

    return convert_kernel


def _make_gather(P, V, D):
    per_w = P // NW
    n_chunks = per_w // CHUNK
    n_groups = n_chunks // NBUF
    assert n_chunks % NBUF == 0 and per_w % CHUNK == 0 and P % NW == 0
    mesh = plsc.VectorSubcoreMesh(core_axis_name="c", subcore_axis_name="s")

    @functools.partial(
        pl.kernel,
        mesh=mesh,
        out_type=jax.ShapeDtypeStruct((P, D), jnp.float32),
        scratch_types=[
            pltpu.VMEM((n_chunks, CHUNK), jnp.int32),
            pltpu.VMEM((NBUF, CHUNK, D), jnp.float32),
            pltpu.SemaphoreType.DMA((NBUF,)),
            pltpu.SemaphoreType.DMA((NBUF,)),
        ],
        compiler_params=pltpu.CompilerParams(use_tc_tiling_on_sc=False),
    )
    def gather_kernel(x_hbm, table_hbm, out_hbm, idx_v, rows_v, gsem, osem):
        wid = lax.axis_index("s") * 2 + lax.axis_index("c")
        base = wid * per_w

        pltpu.sync_copy(x_hbm.at[pl.ds(wid * n_chunks, n_chunks)], idx_v)

        def start_gather(j, b):
            pltpu.async_copy(
                table_hbm.at[idx_v.at[j]], rows_v.at[b], gsem.at[b])

        def wait_gather(j, b):
            pltpu.make_async_copy(
                table_hbm.at[idx_v.at[j]], rows_v.at[b], gsem.at[b]).wait()

        def start_out(j, b):
            pltpu.async_copy(
                rows_v.at[b], out_hbm.at[pl.ds(base + j * CHUNK, CHUNK)],
                osem.at[b])

        def wait_out(j, b):
            pltpu.make_async_copy(
                rows_v.at[b], out_hbm.at[pl.ds(base + j * CHUNK, CHUNK)],
                osem.at[b]).wait()

        for b in range(NBUF):
            start_gather(b, b)
        wait_gather(0, 0)
        start_out(0, 0)

        def group(g, _):
            for b in range(NBUF):
                step = g * NBUF + b
                wait_out(step - NBUF, b)
                start_gather(step, b)
                j_w = step - (NBUF - 1)
                bw = (b + 1) % NBUF
                wait_gather(j_w, bw)
                start_out(j_w, bw)
            return 0

        lax.fori_loop(1, n_groups, group, 0)

        for s in range(NBUF - 1):
            j_w = n_chunks - (NBUF - 1) + s
            wait_gather(j_w, j_w % NBUF)
            start_out(j_w, j_w % NBUF)
        for s in range(NBUF):
            j_w = n_chunks - NBUF + s
            wait_out(j_w, j_w % NBUF)

    return gather_kernel


def kernel(x, table):
    B, L = x.shape
    V, D = table.shape
    P = B * L
    xf = x.reshape(P // CHUNK, CHUNK).astype(jnp.int32)
    tt = table.T
    tlin = _make_convert(V, D)(tt, tt[:, V - VT:])
    out = _make_gather(P, V, D)(xf, tlin.reshape(V, D))
    return out.reshape(B, L, D)


# speedup vs baseline: 1.9245x; 1.9245x over previous
import functools

import jax
import jax.numpy as jnp
from jax import lax
from jax.experimental import pallas as pl
from jax.experimental.pallas import tpu as pltpu
from jax.experimental.pallas import tpu_sc as plsc

NW = 32
CHUNK = 128
NBUF = 4
VT = 256


def _make_convert(V, D):
    n_full = V // VT
    tail = V % VT
    my_n = -(-n_full // NW)
    n_pairs = my_n // 2
    assert my_n % 2 == 1, "schedule below assumes an odd per-worker count"
    mesh = plsc.VectorSubcoreMesh(core_axis_name="c", subcore_axis_name="s")

    @functools.partial(
        pl.kernel,
        mesh=mesh,
        out_type=jax.ShapeDtypeStruct((V * D,), jnp.float32),
        scratch_types=[
            pltpu.VMEM((2 * D, VT), jnp.float32),
            pltpu.VMEM((2 * VT * D,), jnp.float32),
            pltpu.SemaphoreType.DMA((2,)),
            pltpu.SemaphoreType.DMA((2,)),
        ],
        compiler_params=pltpu.CompilerParams(
            use_tc_tiling_on_sc=True, needs_layout_passes=False),
    )
    def convert_kernel(tt_hbm, tail_hbm, out_hbm, in_v, tr_v, isem, osem):
        wid = lax.axis_index("s") * 2 + lax.axis_index("c")
        base = wid * my_n

        lanes = lax.iota(jnp.int32, 16)
        lanes_d = lanes * D

        def v0(s):
            v = jnp.minimum((base + s) * VT, (n_full - 1) * VT)
            return pl.multiple_of(v, VT)

        def start_in(s, b):
            pltpu.async_copy(
                tt_hbm.at[:, pl.ds(v0(s), VT)],
                in_v.at[pl.ds(b * D, D)], isem.at[b])

        def wait_in(b):
            pltpu.make_async_copy(
                tt_hbm.at[:, pl.ds(0, VT)],
                in_v.at[pl.ds(b * D, D)], isem.at[b]).wait()

        def start_out(s, b):
            pltpu.async_copy(
                tr_v.at[pl.ds(b * VT * D, VT * D)],
                out_hbm.at[pl.ds(v0(s) * D, VT * D)], osem.at[b])

        def wait_out(b):
            pltpu.make_async_copy(
                tr_v.at[pl.ds(b * VT * D, VT * D)],
                out_hbm.at[pl.ds(0, VT * D)], osem.at[b]).wait()

        def transpose(b):
            @plsc.parallel_loop(0, D, unroll=2)
            def _(d0):
                dd = d0 + lanes
                dd = jnp.where(dd >= D, dd - D, dd)
                rows = dd + b * D
                obase = lanes_d + dd + b * VT * D
                for k in range(VT // 16):
                    vals = plsc.load_gather(in_v, [rows, k * 16 + lanes])
                    plsc.store_scatter(tr_v, [obase + k * (16 * D)], vals)
            off = b * VT * D + VT * D - 16
            tr_v[pl.ds(off, 16)] = tr_v[pl.ds(off, 16)] + 0.0

        if tail:
            @pl.when(wid == NW - 1)
            def _():
                pltpu.sync_copy(tail_hbm, in_v.at[pl.ds(0, D)])
                @plsc.parallel_loop(0, D, unroll=2)
                def _(d0):
                    dd = d0 + lanes
                    dd = jnp.where(dd >= D, dd - D, dd)
                    obase = lanes_d + dd
                    for k in range(VT // 16):
                        vals = plsc.load_gather(in_v, [dd, k * 16 + lanes])
                        plsc.store_scatter(
                            tr_v, [obase + k * (16 * D)], vals)
                off = VT * D - 16
                tr_v[pl.ds(off, 16)] = tr_v[pl.ds(off, 16)] + 0.0
                pltpu.sync_copy(
                    tr_v.at[pl.ds(0, VT * D)],
                    out_hbm.at[pl.ds((V - VT) * D, VT * D)])

        start_in(0, 0)
        start_in(1, 1)

        def pair(s2, _):
            s0 = s2 * 2
            s1 = s0 + 1

            wait_in(0)

            @pl.when(s2 >= 1)
            def _():
                wait_out(0)

            transpose(0)
            start_out(s0, 0)

            @pl.when(s0 + 2 < my_n)
            def _():
                start_in(s0 + 2, 0)

            @pl.when(s1 < my_n)
            def _():
                wait_in(1)

                @pl.when(s2 >= 1)
                def _():
                    wait_out(1)

                transpose(1)
                start_out(s1, 1)

                @pl.when(s1 + 2 < my_n)
                def _():
                    start_in(s1 + 2, 1)

            return 0

        lax.fori_loop(0, (my_n + 1) // 2, pair, 0)

        wait_out(0)
        wait_out(1)

    return convert_kernel


def _make_gather(P, V, D):
    per_w = P // NW
    n_chunks = per_w // CHUNK
    n_groups = n_chunks // NBUF
    assert n_chunks % NBUF == 0 and per_w % CHUNK == 0 and P % NW == 0
    mesh = plsc.VectorSubcoreMesh(core_axis_name="c", subcore_axis_name="s")

    @functools.partial(
        pl.kernel,
        mesh=mesh,
        out_type=jax.ShapeDtypeStruct((P, D), jnp.float32),
        scratch_types=[
            pltpu.VMEM((n_chunks, CHUNK), jnp.int32),
            pltpu.VMEM((NBUF, CHUNK, D), jnp.float32),
            pltpu.SemaphoreType.DMA((NBUF,)),
            pltpu.SemaphoreType.DMA((NBUF,)),
        ],
        compiler_params=pltpu.CompilerParams(use_tc_tiling_on_sc=False),
    )
    def gather_kernel(x_hbm, table_hbm, out_hbm, idx_v, rows_v, gsem, osem):
        wid = lax.axis_index("s") * 2 + lax.axis_index("c")
        base = wid * per_w

        pltpu.sync_copy(x_hbm.at[pl.ds(wid * n_chunks, n_chunks)], idx_v)

        def start_gather(j, b):
            pltpu.async_copy(
                table_hbm.at[idx_v.at[j]], rows_v.at[b], gsem.at[b])

        def wait_gather(j, b):
            pltpu.make_async_copy(
                table_hbm.at[idx_v.at[j]], rows_v.at[b], gsem.at[b]).wait()

        def start_out(j, b):
            pltpu.async_copy(
                rows_v.at[b], out_hbm.at[pl.ds(base + j * CHUNK, CHUNK)],
                osem.at[b])

        def wait_out(j, b):
            pltpu.make_async_copy(
                rows_v.at[b], out_hbm.at[pl.ds(base + j * CHUNK, CHUNK)],
                osem.at[b]).wait()

        for b in range(NBUF):
            start_gather(b, b)
        wait_gather(0, 0)
        start_out(0, 0)

        def group(g, _):
            for b in range(NBUF):
                step = g * NBUF + b
                wait_out(step - NBUF, b)
                start_gather(step, b)
                j_w = step - (NBUF - 1)
                bw = (b + 1) % NBUF
                wait_gather(j_w, bw)
                start_out(j_w, bw)
            return 0

        lax.fori_loop(1, n_groups, group, 0)

        for s in range(NBUF - 1):
            j_w = n_chunks - (NBUF - 1) + s
            wait_gather(j_w, j_w % NBUF)
            start_out(j_w, j_w % NBUF)
        for s in range(NBUF):
            j_w = n_chunks - NBUF + s
            wait_out(j_w, j_w % NBUF)

    return gather_kernel


def kernel(x, table):
    B, L = x.shape
    V, D = table.shape
    P = B * L
    xf = x.reshape(P // CHUNK, CHUNK).astype(jnp.int32)
    tt = table.T
    tlin = _make_convert(V, D)(tt, tt[:, V - VT:])
    out = _make_gather(P, V, D)(xf, tlin.reshape(V, D))
    return out.reshape(B, L, D)


# speedup vs baseline: 3.0004x; 1.5590x over previous
import functools

import jax
import jax.numpy as jnp
from jax import lax
from jax.experimental import pallas as pl
from jax.experimental.pallas import tpu as pltpu
from jax.experimental.pallas import tpu_sc as plsc

NW = 32
VT = 256
PD = 128


def _make_convert(V, D):
    n_full = V // VT
    tail = V % VT
    my_n = -(-n_full // NW)
    assert my_n % 2 == 1, "schedule below assumes an odd per-worker count"
    mesh = plsc.VectorSubcoreMesh(core_axis_name="c", subcore_axis_name="s")

    @functools.partial(
        pl.kernel,
        mesh=mesh,
        out_type=jax.ShapeDtypeStruct((V * PD,), jnp.float32),
        scratch_types=[
            pltpu.VMEM((2 * D, VT), jnp.float32),
            pltpu.VMEM((2 * VT * PD,), jnp.float32),
            pltpu.SemaphoreType.DMA((2,)),
            pltpu.SemaphoreType.DMA((2,)),
        ],
        compiler_params=pltpu.CompilerParams(
            use_tc_tiling_on_sc=True, needs_layout_passes=False),
    )
    def convert_kernel(tt_hbm, tail_hbm, out_hbm, in_v, tr_v, isem, osem):
        wid = lax.axis_index("s") * 2 + lax.axis_index("c")
        base = wid * my_n

        lanes = lax.iota(jnp.int32, 16)
        lanes_p = lanes * PD

        def v0(s):
            v = jnp.minimum((base + s) * VT, (n_full - 1) * VT)
            return pl.multiple_of(v, VT)

        def start_in(s, b):
            pltpu.async_copy(
                tt_hbm.at[:, pl.ds(v0(s), VT)],
                in_v.at[pl.ds(b * D, D)], isem.at[b])

        def wait_in(b):
            pltpu.make_async_copy(
                tt_hbm.at[:, pl.ds(0, VT)],
                in_v.at[pl.ds(b * D, D)], isem.at[b]).wait()

        def start_out(s, b):
            pltpu.async_copy(
                tr_v.at[pl.ds(b * VT * PD, VT * PD)],
                out_hbm.at[pl.ds(v0(s) * PD, VT * PD)], osem.at[b])

        def wait_out(b):
            pltpu.make_async_copy(
                tr_v.at[pl.ds(b * VT * PD, VT * PD)],
                out_hbm.at[pl.ds(0, VT * PD)], osem.at[b]).wait()

        def transpose(b):
            @plsc.parallel_loop(0, D, unroll=2)
            def _(d0):
                dd = d0 + lanes
                dd = jnp.where(dd >= D, dd - D, dd)
                rows = dd + b * D
                obase = lanes_p + dd + b * VT * PD
                for k in range(VT // 16):
                    vals = plsc.load_gather(in_v, [rows, k * 16 + lanes])
                    plsc.store_scatter(tr_v, [obase + k * (16 * PD)], vals)
            off = b * VT * PD + VT * PD - 16
            tr_v[pl.ds(off, 16)] = tr_v[pl.ds(off, 16)] + 0.0

        if tail:
            @pl.when(wid == NW - 1)
            def _():
                pltpu.sync_copy(tail_hbm, in_v.at[pl.ds(0, D)])
                @plsc.parallel_loop(0, D, unroll=2)
                def _(d0):
                    dd = d0 + lanes
                    dd = jnp.where(dd >= D, dd - D, dd)
                    obase = lanes_p + dd
                    for k in range(VT // 16):
                        vals = plsc.load_gather(in_v, [dd, k * 16 + lanes])
                        plsc.store_scatter(
                            tr_v, [obase + k * (16 * PD)], vals)
                off = VT * PD - 16
                tr_v[pl.ds(off, 16)] = tr_v[pl.ds(off, 16)] + 0.0
                pltpu.sync_copy(
                    tr_v.at[pl.ds(0, VT * PD)],
                    out_hbm.at[pl.ds((V - VT) * PD, VT * PD)])

        start_in(0, 0)
        start_in(1, 1)

        def pair(s2, _):
            s0 = s2 * 2
            s1 = s0 + 1

            wait_in(0)

            @pl.when(s2 >= 1)
            def _():
                wait_out(0)

            transpose(0)
            start_out(s0, 0)

            @pl.when(s0 + 2 < my_n)
            def _():
                start_in(s0 + 2, 0)

            @pl.when(s1 < my_n)
            def _():
                wait_in(1)

                @pl.when(s2 >= 1)
                def _():
                    wait_out(1)

                transpose(1)
                start_out(s1, 1)

                @pl.when(s1 + 2 < my_n)
                def _():
                    start_in(s1 + 2, 1)

            return 0

        lax.fori_loop(0, (my_n + 1) // 2, pair, 0)

        wait_out(0)
        wait_out(1)

    return convert_kernel


def _make_gather(B, L, V, D):
    bpw = B // NW
    nbb = bpw // 128
    n_chunks = L * nbb
    assert n_chunks % 2 == 0
    mesh = plsc.VectorSubcoreMesh(core_axis_name="c", subcore_axis_name="s")

    @functools.partial(
        pl.kernel,
        mesh=mesh,
        out_type=jax.ShapeDtypeStruct((L, D, B), jnp.float32),
        scratch_types=[
            pltpu.VMEM((L, bpw), jnp.int32),
            pltpu.VMEM((2 * 128, PD), jnp.float32),
            pltpu.VMEM((2, D, 128), jnp.float32),
            pltpu.SemaphoreType.DMA((2,)),
            pltpu.SemaphoreType.DMA((2,)),
        ],
        compiler_params=pltpu.CompilerParams(
            use_tc_tiling_on_sc=True, needs_layout_passes=False),
    )
    def gather_kernel(xt_hbm, table_hbm, out_hbm, idx_v, rows_v, tr_v,
                      gsem, osem):
        wid = lax.axis_index("s") * 2 + lax.axis_index("c")
        b_base = wid * bpw

        lanes = lax.iota(jnp.int32, 16)
        zeros = lanes * 0

        pltpu.sync_copy(xt_hbm.at[:, pl.ds(b_base, bpw)], idx_v)

        def lbb(j):
            return j // nbb, lax.rem(j, nbb)

        def start_gather(j, h):
            l, bb = lbb(j)
            pltpu.async_copy(
                table_hbm.at[idx_v.at[l, pl.ds(bb * 128, 128)]],
                rows_v.at[pl.ds(h * 128, 128)], gsem.at[h])

        def wait_gather(h):
            pltpu.make_async_copy(
                table_hbm.at[idx_v.at[0, pl.ds(0, 128)]],
                rows_v.at[pl.ds(h * 128, 128)], gsem.at[h]).wait()

        def start_out(j, h):
            l, bb = lbb(j)
            pltpu.async_copy(
                tr_v.at[pl.ds(h, 1)],
                out_hbm.at[pl.ds(l, 1), :,
                           pl.ds(b_base + bb * 128, 128)], osem.at[h])

        def wait_out(h):
            pltpu.make_async_copy(
                tr_v.at[pl.ds(h, 1)],
                out_hbm.at[pl.ds(0, 1), :, pl.ds(0, 128)], osem.at[h]).wait()

        def transpose(h):
            @plsc.parallel_loop(0, D, unroll=2)
            def _(d0):
                dd = d0 + lanes
                dd = jnp.where(dd >= D, dd - D, dd)
                hv = zeros + h
                for k in range(128 // 16):
                    ivec = k * 16 + lanes
                    vals = plsc.load_gather(
                        rows_v, [h * 128 + ivec, dd])
                    plsc.store_scatter(tr_v, [hv, dd, ivec], vals)
            off = 128 - 16
            tr_v[h, D - 1, pl.ds(off, 16)] = (
                tr_v[h, D - 1, pl.ds(off, 16)] + 0.0)

        start_gather(0, 0)
        start_gather(1, 1)

        def pair(p, _):
            j0 = p * 2
            for h in range(2):
                j = j0 + h
                wait_gather(h)

                @pl.when(p >= 1)
                def _():
                    wait_out(h)

                transpose(h)
                start_out(j, h)

                @pl.when(j + 2 < n_chunks)
                def _():
                    start_gather(j + 2, h)
            return 0

        lax.fori_loop(0, n_chunks // 2, pair, 0)

        wait_out(0)
        wait_out(1)

    return gather_kernel


def kernel(x, table):
    B, L = x.shape
    V, D = table.shape
    tt = table.T
    tlin = _make_convert(V, D)(tt, tt[:, V - VT:])
    out3 = _make_gather(B, L, V, D)(
        x.T.astype(jnp.int32), tlin.reshape(V, PD))
    return jnp.transpose(out3, (2, 0, 1))
